# baseline (speedup 1.0000x reference)
import dataclasses
import functools

import jax
import jax.numpy as jnp
from jax import lax
from jax.experimental import pallas as pl
from jax.experimental.pallas import tpu as pltpu
from jax.experimental.pallas import tpu_sc as plsc

_K = 256
_N = 32768
_B = 128
_SCALE = 128.0
_RB = 8
_NW = 32
_RPW = _B // _NW
_QCAP = 4096
_QSTRIDE = _QCAP + 16
_SAMP = 64


def _monokey(x):
    u = lax.bitcast_convert_type(x, jnp.uint32)
    return jnp.where(u >= jnp.uint32(0x80000000), ~u,
                     u | jnp.uint32(0x80000000))


def _ikey(x):
    b = lax.bitcast_convert_type(x, jnp.int32)
    return b ^ ((b >> 31) & jnp.int32(0x7FFFFFFF))


def _ikey_inv(ik):
    b = jnp.where(ik < 0, ik ^ jnp.int32(0x7FFFFFFF), ik)
    return lax.bitcast_convert_type(b, jnp.float32)


def _tie_index_cutoff(eq, t, r):
    idx = lax.broadcasted_iota(jnp.int32, eq.shape, 1)
    lo0 = jnp.zeros((r, 1), jnp.int32)
    hi0 = jnp.full((r, 1), _N, jnp.int32)

    def bs_idx(_, carry):
        lo, hi = carry
        mid = (lo + hi) // 2
        cnt = jnp.sum((eq & (idx >= mid)).astype(jnp.int32), axis=1,
                      keepdims=True)
        ge = cnt >= t
        return jnp.where(ge, mid, lo), jnp.where(ge, hi, mid)

    lo, _ = lax.fori_loop(0, 15, bs_idx, (lo0, hi0))
    return lo


def _exact_mask(x):
    r = x.shape[0]
    key = _monokey(x)

    lo0 = jnp.zeros((r, 1), jnp.uint32)
    hi0 = jnp.full((r, 1), 0xFFFFFFFF, jnp.uint32)

    def bs_val(_, carry):
        lo, hi = carry
        mid = lo + (hi - lo) // 2
        cnt = jnp.sum((key > mid).astype(jnp.int32), axis=1, keepdims=True)
        ge = cnt >= _K
        return jnp.where(ge, mid + 1, lo), jnp.where(ge, hi, mid)

    lo, _ = lax.fori_loop(0, 32, bs_val, (lo0, hi0))
    thresh = lo

    gt = key > thresh
    c = jnp.sum(gt.astype(jnp.int32), axis=1, keepdims=True)
    t = _K - c
    eq = key == thresh
    cnt_eq = jnp.sum(eq.astype(jnp.int32), axis=1, keepdims=True)

    istar = lax.cond(jnp.any(cnt_eq > t),
                     lambda _: _tie_index_cutoff(eq, t, r),
                     lambda _: jnp.zeros((r, 1), jnp.int32), None)
    idx = lax.broadcasted_iota(jnp.int32, x.shape, 1)
    mask = gt | (eq & (idx >= istar))
    return jnp.where(mask, x * _SCALE, 0.0)


def _tc_mask_body(feat_ref, par_ref, out_ref):
    x = feat_ref[...]
    p = par_ref[...]
    r = x.shape[0]

    def fast(_):
        tf = p[:, 0:1]
        t = p[:, 1:2].astype(jnp.int32)
        cnt_eq = p[:, 3:4].astype(jnp.int32)
        gt = x > tf
        eq = x == tf
        istar = lax.cond(jnp.any(cnt_eq > t),
                         lambda _: _tie_index_cutoff(eq, t, r),
                         lambda _: jnp.zeros((r, 1), jnp.int32), None)
        idx = lax.broadcasted_iota(jnp.int32, x.shape, 1)
        mask = gt | (eq & (idx >= istar))
        return jnp.where(mask, x * _SCALE, 0.0)

    all_valid = jnp.all(p[:, 2:3] > 0.5)
    out_ref[...] = lax.cond(all_valid, fast, lambda _: _exact_mask(x), None)


def _sc_body(feat_hbm, par_hbm, row_v, key_v, par_v, sem):
    cid = lax.axis_index("c")
    sid = lax.axis_index("s")
    base = (sid * 2 + cid) * _RPW
    zero16i = jnp.zeros((16,), jnp.int32)
    lane = lax.iota(jnp.int32, 16)

    for rr in range(_RPW):
        row = base + rr
        pltpu.async_copy(feat_hbm.at[row], row_v, sem).wait()

        def stat_body(i, carry):
            s1, s2 = carry
            v = row_v[pl.ds(i * (_N // _SAMP), 16)]
            return s1 + v, s2 + v * v

        s1, s2 = lax.fori_loop(0, _SAMP, stat_body,
                               (jnp.zeros((16,), jnp.float32),
                                jnp.zeros((16,), jnp.float32)))
        inv = jnp.float32(1.0 / (16 * _SAMP))
        mu = jnp.sum(s1) * inv
        var = jnp.minimum(jnp.maximum(jnp.sum(s2) * inv - mu * mu,
                                      jnp.float32(1e-12)), jnp.float32(2.0))
        var_v = jnp.broadcast_to(var, (16,))
        y = lax.fori_loop(
            0, 12, lambda i, y: y * (1.5 - 0.5 * var_v * y * y),
            jnp.ones((16,), jnp.float32))
        sig_v = var_v * y
        thr_v = jnp.broadcast_to(mu, (16,)) + 2.0 * sig_v

        qn = _N // 16
        thrkey_v = _ikey(thr_v)
        qcap_v = jnp.full((16,), _QCAP, jnp.int32)

        def filt(i, carry):
            ptr_v, mx = carry
            for u in range(4):
                v = row_v[pl.ds((i * 4 + u) * 16, 16)]
                k = _ikey(v)
                mx = jnp.maximum(mx, k)
                m = v > thr_v
                off = plsc.cumsum(m.astype(jnp.int32)) - 1
                idx = jnp.minimum(ptr_v + off, qcap_v)
                plsc.store_scatter(key_v, [idx], k, mask=m)
                ptr_v = ptr_v + plsc.all_reduce_population_count(m)
            return ptr_v, mx

        ptr_v, maxkey_v = lax.fori_loop(
            0, qn // 4, filt,
            (jnp.zeros((16,), jnp.int32),
             jnp.full((16,), -0x80000000, jnp.int32)))
        c_total = jnp.max(ptr_v)
        maxkey_v = jnp.broadcast_to(jnp.max(maxkey_v), (16,))
        valid = (c_total >= _K) & (c_total <= _QCAP)
        pad16 = jnp.full((16,), -0x80000000, jnp.int32)
        pc = jnp.minimum(c_total, _QCAP)
        for z in range(4):
            plsc.store_compressed(key_v.at[pl.ds(pc + z * 16, 16)],
                                  pad16, mask=pad16 < 0)
        nch4 = (pc + 63) >> 6

        def count_gt(mid_v):
            def cb(j, acc):
                for u in range(4):
                    k = key_v[pl.ds(j * 64 + u * 16, 16)]
                    acc = acc + (k > mid_v).astype(jnp.int32)
                return acc
            return jnp.sum(lax.fori_loop(0, nch4, cb, zero16i))

        def bs(i, lohi):
            lo, hi = lohi
            mid = (lo >> 1) + (hi >> 1) + (lo & hi & 1)
            big = jnp.broadcast_to(count_gt(mid) >= _K, (16,))
            return (jnp.where(big, mid + 1, lo), jnp.where(big, hi, mid))

        lo, hi = lax.fori_loop(0, 27, bs, (thrkey_v, maxkey_v))
        tkey_v = lo
        valid = valid & jnp.all(lo == hi)

        def cnt2(j, carry):
            aa, ee = carry
            for u in range(4):
                k = key_v[pl.ds(j * 64 + u * 16, 16)]
                aa = aa + (k > tkey_v).astype(jnp.int32)
                ee = ee + (k == tkey_v).astype(jnp.int32)
            return aa, ee

        a, e = lax.fori_loop(0, nch4, cnt2, (zero16i, zero16i))
        c_above = jnp.sum(a)
        cnt_eq = jnp.sum(e)
        t = _K - c_above

        tf_v = _ikey_inv(tkey_v)

        pvec = jnp.where(lane == 0, tf_v, 0.0)
        pvec = jnp.where(lane == 1,
                         jnp.broadcast_to(t.astype(jnp.float32), (16,)), pvec)
        pvec = jnp.where(lane == 2,
                         jnp.broadcast_to(
                             jnp.where(valid, jnp.float32(1.0),
                                       jnp.float32(0.0)), (16,)), pvec)
        pvec = jnp.where(lane == 3,
                         jnp.broadcast_to(cnt_eq.astype(jnp.float32), (16,)),
                         pvec)
        par_v[rr, :] = pvec

    pltpu.async_copy(par_v, par_hbm.at[pl.ds(base, _RPW)], sem).wait()


def _sc_params(feat):
    mesh = plsc.VectorSubcoreMesh(core_axis_name="c", subcore_axis_name="s",
                                  num_cores=2, num_subcores=16)
    cp = pltpu.CompilerParams()
    if "needs_layout_passes" in pltpu.CompilerParams.__dataclass_fields__:
        cp = dataclasses.replace(cp, needs_layout_passes=False)
    return pl.kernel(
        _sc_body,
        compiler_params=cp,
        out_type=jax.ShapeDtypeStruct((_B, 16), jnp.float32),
        mesh=mesh,
        scratch_types=[
            pltpu.VMEM((_N,), jnp.float32),
            pltpu.VMEM((4 * _QSTRIDE,), jnp.int32),
            pltpu.VMEM((_RPW, 16), jnp.float32),
            pltpu.SemaphoreType.DMA,
        ],
    )(feat)


def kernel(feat):
    b, n = feat.shape
    params = _sc_params(feat)
    return pl.pallas_call(
        _tc_mask_body,
        grid=(b // _RB,),
        in_specs=[
            pl.BlockSpec((_RB, n), lambda i: (i, 0)),
            pl.BlockSpec((_RB, 16), lambda i: (i, 0)),
        ],
        out_specs=pl.BlockSpec((_RB, n), lambda i: (i, 0)),
        out_shape=jax.ShapeDtypeStruct(feat.shape, feat.dtype),
    )(feat, params)


# speedup vs baseline: 1.0359x; 1.0359x over previous
import dataclasses
import functools

import jax
import jax.numpy as jnp
from jax import lax
from jax.experimental import pallas as pl
from jax.experimental.pallas import tpu as pltpu
from jax.experimental.pallas import tpu_sc as plsc

_K = 256
_N = 32768
_B = 128
_SCALE = 128.0
_RB = 8
_NW = 32
_RPW = _B // _NW
_QCAP = 4096
_QSTRIDE = _QCAP + 16
_SAMP = 64


def _monokey(x):
    u = lax.bitcast_convert_type(x, jnp.uint32)
    return jnp.where(u >= jnp.uint32(0x80000000), ~u,
                     u | jnp.uint32(0x80000000))


def _ikey(x):
    b = lax.bitcast_convert_type(x, jnp.int32)
    return b ^ ((b >> 31) & jnp.int32(0x7FFFFFFF))


def _ikey_inv(ik):
    b = jnp.where(ik < 0, ik ^ jnp.int32(0x7FFFFFFF), ik)
    return lax.bitcast_convert_type(b, jnp.float32)


def _tie_index_cutoff(eq, t, r):
    idx = lax.broadcasted_iota(jnp.int32, eq.shape, 1)
    lo0 = jnp.zeros((r, 1), jnp.int32)
    hi0 = jnp.full((r, 1), _N, jnp.int32)

    def bs_idx(_, carry):
        lo, hi = carry
        mid = (lo + hi) // 2
        cnt = jnp.sum((eq & (idx >= mid)).astype(jnp.int32), axis=1,
                      keepdims=True)
        ge = cnt >= t
        return jnp.where(ge, mid, lo), jnp.where(ge, hi, mid)

    lo, _ = lax.fori_loop(0, 15, bs_idx, (lo0, hi0))
    return lo


def _exact_mask(x):
    r = x.shape[0]
    key = _monokey(x)

    lo0 = jnp.zeros((r, 1), jnp.uint32)
    hi0 = jnp.full((r, 1), 0xFFFFFFFF, jnp.uint32)

    def bs_val(_, carry):
        lo, hi = carry
        mid = lo + (hi - lo) // 2
        cnt = jnp.sum((key > mid).astype(jnp.int32), axis=1, keepdims=True)
        ge = cnt >= _K
        return jnp.where(ge, mid + 1, lo), jnp.where(ge, hi, mid)

    lo, _ = lax.fori_loop(0, 32, bs_val, (lo0, hi0))
    thresh = lo

    gt = key > thresh
    c = jnp.sum(gt.astype(jnp.int32), axis=1, keepdims=True)
    t = _K - c
    eq = key == thresh
    cnt_eq = jnp.sum(eq.astype(jnp.int32), axis=1, keepdims=True)

    istar = lax.cond(jnp.any(cnt_eq > t),
                     lambda _: _tie_index_cutoff(eq, t, r),
                     lambda _: jnp.zeros((r, 1), jnp.int32), None)
    idx = lax.broadcasted_iota(jnp.int32, x.shape, 1)
    mask = gt | (eq & (idx >= istar))
    return jnp.where(mask, x * _SCALE, 0.0)


def _tc_mask_body(feat_ref, par_ref, out_ref):
    x = feat_ref[...]
    p = par_ref[...]
    r = x.shape[0]

    def fast(_):
        tf = p[:, 0:1]
        t = p[:, 1:2].astype(jnp.int32)
        cnt_eq = p[:, 3:4].astype(jnp.int32)
        gt = x > tf
        eq = x == tf
        istar = lax.cond(jnp.any(cnt_eq > t),
                         lambda _: _tie_index_cutoff(eq, t, r),
                         lambda _: jnp.zeros((r, 1), jnp.int32), None)
        idx = lax.broadcasted_iota(jnp.int32, x.shape, 1)
        mask = gt | (eq & (idx >= istar))
        return jnp.where(mask, x * _SCALE, 0.0)

    all_valid = jnp.all(p[:, 2:3] > 0.5)
    out_ref[...] = lax.cond(all_valid, fast, lambda _: _exact_mask(x), None)


def _sc_body(feat_hbm, par_hbm, row_a, row_b, key_v, par_v, sem_a, sem_b):
    cid = lax.axis_index("c")
    sid = lax.axis_index("s")
    base = (sid * 2 + cid) * _RPW
    zero16i = jnp.zeros((16,), jnp.int32)
    lane = lax.iota(jnp.int32, 16)

    bufs = (row_a, row_b)
    sems = (sem_a, sem_b)
    pending = pltpu.async_copy(feat_hbm.at[base], row_a, sem_a)
    for rr in range(_RPW):
        row_v = bufs[rr % 2]
        pending.wait()
        if rr + 1 < _RPW:
            pending = pltpu.async_copy(feat_hbm.at[base + rr + 1],
                                       bufs[(rr + 1) % 2],
                                       sems[(rr + 1) % 2])

        def stat_body(i, carry):
            s1, s2 = carry
            v = row_v[pl.ds(i * (_N // _SAMP), 16)]
            return s1 + v, s2 + v * v

        s1, s2 = lax.fori_loop(0, _SAMP, stat_body,
                               (jnp.zeros((16,), jnp.float32),
                                jnp.zeros((16,), jnp.float32)))
        inv = jnp.float32(1.0 / (16 * _SAMP))
        mu = jnp.sum(s1) * inv
        var = jnp.minimum(jnp.maximum(jnp.sum(s2) * inv - mu * mu,
                                      jnp.float32(1e-12)), jnp.float32(2.0))
        var_v = jnp.broadcast_to(var, (16,))
        y = lax.fori_loop(
            0, 12, lambda i, y: y * (1.5 - 0.5 * var_v * y * y),
            jnp.ones((16,), jnp.float32))
        sig_v = var_v * y
        thr_v = jnp.broadcast_to(mu, (16,)) + 2.0 * sig_v

        qn = _N // 16
        thrkey_v = _ikey(thr_v)
        qcap_v = jnp.full((16,), _QCAP, jnp.int32)

        def filt(i, carry):
            ptr_v, mx = carry
            for u in range(4):
                v = row_v[pl.ds((i * 4 + u) * 16, 16)]
                k = _ikey(v)
                mx = jnp.maximum(mx, k)
                m = v > thr_v
                off = plsc.cumsum(m.astype(jnp.int32)) - 1
                idx = jnp.minimum(ptr_v + off, qcap_v)
                plsc.store_scatter(key_v, [idx], k, mask=m)
                ptr_v = ptr_v + plsc.all_reduce_population_count(m)
            return ptr_v, mx

        ptr_v, maxkey_v = lax.fori_loop(
            0, qn // 4, filt,
            (jnp.zeros((16,), jnp.int32),
             jnp.full((16,), -0x80000000, jnp.int32)))
        c_total = jnp.max(ptr_v)
        maxkey_v = jnp.broadcast_to(jnp.max(maxkey_v), (16,))
        valid = (c_total >= _K) & (c_total <= _QCAP)
        pad16 = jnp.full((16,), -0x80000000, jnp.int32)
        pc = jnp.minimum(c_total, _QCAP)
        for z in range(4):
            plsc.store_compressed(key_v.at[pl.ds(pc + z * 16, 16)],
                                  pad16, mask=pad16 < 0)
        nch4 = (pc + 63) >> 6

        def count_gt(mid_v):
            def cb(j, acc):
                for u in range(4):
                    k = key_v[pl.ds(j * 64 + u * 16, 16)]
                    acc = acc + (k > mid_v).astype(jnp.int32)
                return acc
            return jnp.sum(lax.fori_loop(0, nch4, cb, zero16i))

        def bs(i, lohi):
            lo, hi = lohi
            mid = (lo >> 1) + (hi >> 1) + (lo & hi & 1)
            big = jnp.broadcast_to(count_gt(mid) >= _K, (16,))
            return (jnp.where(big, mid + 1, lo), jnp.where(big, hi, mid))

        lo, hi = lax.fori_loop(0, 27, bs, (thrkey_v, maxkey_v))
        tkey_v = lo
        valid = valid & jnp.all(lo == hi)

        def cnt2(j, carry):
            aa, ee = carry
            for u in range(4):
                k = key_v[pl.ds(j * 64 + u * 16, 16)]
                aa = aa + (k > tkey_v).astype(jnp.int32)
                ee = ee + (k == tkey_v).astype(jnp.int32)
            return aa, ee

        a, e = lax.fori_loop(0, nch4, cnt2, (zero16i, zero16i))
        c_above = jnp.sum(a)
        cnt_eq = jnp.sum(e)
        t = _K - c_above

        tf_v = _ikey_inv(tkey_v)

        pvec = jnp.where(lane == 0, tf_v, 0.0)
        pvec = jnp.where(lane == 1,
                         jnp.broadcast_to(t.astype(jnp.float32), (16,)), pvec)
        pvec = jnp.where(lane == 2,
                         jnp.broadcast_to(
                             jnp.where(valid, jnp.float32(1.0),
                                       jnp.float32(0.0)), (16,)), pvec)
        pvec = jnp.where(lane == 3,
                         jnp.broadcast_to(cnt_eq.astype(jnp.float32), (16,)),
                         pvec)
        par_v[rr, :] = pvec

    pltpu.async_copy(par_v, par_hbm.at[pl.ds(base, _RPW)], sem_a).wait()


def _sc_params(feat):
    mesh = plsc.VectorSubcoreMesh(core_axis_name="c", subcore_axis_name="s",
                                  num_cores=2, num_subcores=16)
    cp = pltpu.CompilerParams()
    if "needs_layout_passes" in pltpu.CompilerParams.__dataclass_fields__:
        cp = dataclasses.replace(cp, needs_layout_passes=False)
    return pl.kernel(
        _sc_body,
        compiler_params=cp,
        out_type=jax.ShapeDtypeStruct((_B, 16), jnp.float32),
        mesh=mesh,
        scratch_types=[
            pltpu.VMEM((_N,), jnp.float32),
            pltpu.VMEM((_N,), jnp.float32),
            pltpu.VMEM((4 * _QSTRIDE,), jnp.int32),
            pltpu.VMEM((_RPW, 16), jnp.float32),
            pltpu.SemaphoreType.DMA,
            pltpu.SemaphoreType.DMA,
        ],
    )(feat)


def kernel(feat):
    b, n = feat.shape
    params = _sc_params(feat)
    return pl.pallas_call(
        _tc_mask_body,
        grid=(b // _RB,),
        in_specs=[
            pl.BlockSpec((_RB, n), lambda i: (i, 0)),
            pl.BlockSpec((_RB, 16), lambda i: (i, 0)),
        ],
        out_specs=pl.BlockSpec((_RB, n), lambda i: (i, 0)),
        out_shape=jax.ShapeDtypeStruct(feat.shape, feat.dtype),
    )(feat, params)
